# Initial kernel scaffold; baseline (speedup 1.0000x reference)
import jax
import jax.numpy as jnp
from jax.experimental import pallas as pl


def kernel(feat_c0, feat_c1, idx_c01, idx_c10):
    raise NotImplementedError("write your pallas kernel here")


# speedup vs baseline: 4.5605x; 4.5605x over previous
import jax
import jax.numpy as jnp
from jax import lax
from jax.experimental import pallas as pl
from jax.experimental.pallas import tpu as pltpu
from jax.experimental.pallas import tpu_sc as plsc

L = 9216
C = 128
K = 64
LANES = 16
NC, NS = 2, 16
NW = NC * NS
RPW = L // NW
SCALE = 10.0
INV_K = 1.0 / K
NGROUPS = K // LANES
CCHUNKS = C // LANES
IDX_BITS = 14
BIG_KEY = 1 << 22

_mesh = plsc.VectorSubcoreMesh(
    core_axis_name="c", subcore_axis_name="s", num_cores=NC, num_subcores=NS
)


def _match_body(f0_hbm, f1_hbm, i01_hbm, i10_hbm,
                conf_hbm, nc01_hbm, ni01_hbm, nc10_hbm, ni10_hbm,
                idxb, fqb, rows, confb, ncb, nib, sem):
    wid = lax.axis_index("s") * NC + lax.axis_index("c")
    base = wid * RPW
    kiota = lax.iota(jnp.int32, LANES)
    lane0 = kiota == 0

    def run_direction(fq_hbm, fc_hbm, idx_hbm, store_conf):
        pltpu.sync_copy(idx_hbm.at[pl.ds(base, RPW)], idxb)
        pltpu.sync_copy(fq_hbm.at[pl.ds(base, RPW)], fqb)

        kflat = [(kiota + g * LANES) * C for g in range(NGROUPS)]

        def row_body(r, carry):
            pltpu.async_copy(fc_hbm.at[idxb.at[r]], rows, sem).wait()

            accs = tuple(jnp.zeros((LANES,), jnp.float32)
                         for _ in range(NGROUPS))
            for chunk in range(CCHUNKS):
                f0c = fqb[r, pl.ds(chunk * LANES, LANES)]

                def c_body(j, accs, f0c=f0c, chunk=chunk):
                    jvec = jnp.full((LANES,), j, jnp.int32)
                    svec = f0c.at[jvec].get(mode="promise_in_bounds")
                    c = jvec + (chunk * LANES)
                    return tuple(
                        accs[g]
                        + plsc.load_gather(rows, [kiota + g * LANES, c])
                        * svec
                        for g in range(NGROUPS)
                    )

                accs = lax.fori_loop(0, LANES, c_body, accs)

            sims = [a * SCALE for a in accs]
            m = jnp.max(jnp.maximum(jnp.maximum(sims[0], sims[1]),
                                    jnp.maximum(sims[2], sims[3])))
            es = [jnp.exp(sv - m) for sv in sims]
            ssum = jnp.sum((es[0] + es[1]) + (es[2] + es[3]))
            invv = 1.0 / jnp.full((LANES,), ssum)
            mkey = BIG_KEY
            for g in range(NGROUPS):
                iv = idxb[r, pl.ds(g * LANES, LANES)]
                key = ((kiota + g * LANES) << IDX_BITS) | iv
                mkey = jnp.minimum(mkey,
                                   jnp.min(jnp.where(sims[g] >= m, key,
                                                     BIG_KEY)))
            nidx = mkey & ((1 << IDX_BITS) - 1)
            if store_conf:
                for g in range(NGROUPS):
                    confb[r, pl.ds(g * LANES, LANES)] = es[g] * invv
            rvec = jnp.full((LANES,), r, jnp.int32)
            plsc.store_scatter(ncb, [rvec], invv, mask=lane0)
            plsc.store_scatter(nib, [rvec], jnp.full((LANES,), nidx),
                               mask=lane0)
            return carry

        lax.fori_loop(0, RPW, row_body, 0)

    run_direction(f0_hbm, f1_hbm, i01_hbm, True)
    pltpu.sync_copy(confb, conf_hbm.at[pl.ds(base, RPW)])
    pltpu.sync_copy(ncb, nc01_hbm.at[pl.ds(base, RPW)])
    pltpu.sync_copy(nib, ni01_hbm.at[pl.ds(base, RPW)])

    run_direction(f1_hbm, f0_hbm, i10_hbm, False)
    pltpu.sync_copy(ncb, nc10_hbm.at[pl.ds(base, RPW)])
    pltpu.sync_copy(nib, ni10_hbm.at[pl.ds(base, RPW)])


_match = pl.kernel(
    _match_body,
    out_type=[
        jax.ShapeDtypeStruct((L, K), jnp.float32),
        jax.ShapeDtypeStruct((L,), jnp.float32),
        jax.ShapeDtypeStruct((L,), jnp.int32),
        jax.ShapeDtypeStruct((L,), jnp.float32),
        jax.ShapeDtypeStruct((L,), jnp.int32),
    ],
    mesh=_mesh,
    scratch_types=[
        pltpu.VMEM((RPW, K), jnp.int32),
        pltpu.VMEM((RPW, C), jnp.float32),
        pltpu.VMEM((K, C), jnp.float32),
        pltpu.VMEM((RPW, K), jnp.float32),
        pltpu.VMEM((RPW,), jnp.float32),
        pltpu.VMEM((RPW,), jnp.int32),
        pltpu.SemaphoreType.DMA,
    ],
    compiler_params=pltpu.CompilerParams(needs_layout_passes=False),
)


def _mutual_body(nc01_hbm, ni01_hbm, ni10_hbm, mconf_hbm,
                 ni10_full, ni01b, nc01b, mb):
    wid = lax.axis_index("s") * NC + lax.axis_index("c")
    base = wid * RPW
    kiota = lax.iota(jnp.int32, LANES)

    pltpu.sync_copy(ni10_hbm, ni10_full)
    pltpu.sync_copy(ni01_hbm.at[pl.ds(base, RPW)], ni01b)
    pltpu.sync_copy(nc01_hbm.at[pl.ds(base, RPW)], nc01b)

    def j_body(j, carry):
        off = j * LANES
        idxv = ni01b[pl.ds(off, LANES)]
        back = plsc.load_gather(ni10_full, [idxv])
        cv = nc01b[pl.ds(off, LANES)]
        lvec = base + off + kiota
        mask = (cv > INV_K) & (back == lvec)
        mb[pl.ds(off, LANES)] = jnp.where(mask, cv, 0.0)
        return carry

    lax.fori_loop(0, RPW // LANES, j_body, 0)
    pltpu.sync_copy(mb, mconf_hbm.at[pl.ds(base, RPW)])


_mutual = pl.kernel(
    _mutual_body,
    out_type=[jax.ShapeDtypeStruct((L,), jnp.float32)],
    mesh=_mesh,
    scratch_types=[
        pltpu.VMEM((L,), jnp.int32),
        pltpu.VMEM((RPW,), jnp.int32),
        pltpu.VMEM((RPW,), jnp.float32),
        pltpu.VMEM((RPW,), jnp.float32),
    ],
    compiler_params=pltpu.CompilerParams(needs_layout_passes=False),
)


def kernel(feat_c0, feat_c1, idx_c01, idx_c10):
    f0 = lax.optimization_barrier(
        (feat_c0[0] / (C ** 0.5)).astype(jnp.bfloat16)).astype(jnp.float32)
    f1 = lax.optimization_barrier(
        (feat_c1[0] / (C ** 0.5)).astype(jnp.bfloat16)).astype(jnp.float32)
    i01, i10 = idx_c01[0], idx_c10[0]
    conf, nc01, ni01, nc10, ni10 = _match(f0, f1, i01, i10)
    (mconf,) = _mutual(nc01, ni01, ni10)
    return (conf[None], nc01[None], ni01[None], nc10[None], ni10[None],
            mconf[None])


# speedup vs baseline: 5.0751x; 1.1128x over previous
import jax
import jax.numpy as jnp
from jax import lax
from jax.experimental import pallas as pl
from jax.experimental.pallas import tpu as pltpu
from jax.experimental.pallas import tpu_sc as plsc

L = 9216
C = 128
K = 64
LANES = 16
NC, NS = 2, 16
NW = NC * NS
RPW = L // NW
SCALE = 10.0
INV_K = 1.0 / K
NGROUPS = K // LANES
CCHUNKS = C // LANES
IDX_BITS = 14
BIG_KEY = 1 << 22

_mesh = plsc.VectorSubcoreMesh(
    core_axis_name="c", subcore_axis_name="s", num_cores=NC, num_subcores=NS
)


def _match_body(f0_hbm, f1_hbm, i01_hbm, i10_hbm,
                conf_hbm, nc01_hbm, ni01_hbm, nc10_hbm, ni10_hbm,
                idxb, fqb, rows0, rows1, confb, ncb, nib, sem0, sem1):
    wid = lax.axis_index("s") * NC + lax.axis_index("c")
    base = wid * RPW
    kiota = lax.iota(jnp.int32, LANES)
    lane0 = kiota == 0

    def run_direction(fq_hbm, fc_hbm, idx_hbm, store_conf):
        pltpu.sync_copy(idx_hbm.at[pl.ds(base * K, RPW * K)], idxb)
        pltpu.sync_copy(fq_hbm.at[pl.ds(base, RPW)], fqb)

        def compute_row(r, rows):
            accs = tuple(jnp.zeros((LANES,), jnp.float32)
                         for _ in range(NGROUPS))
            for chunk in range(CCHUNKS):
                f0c = fqb[r, pl.ds(chunk * LANES, LANES)]

                def c_body(j, accs, f0c=f0c, chunk=chunk):
                    jvec = jnp.full((LANES,), j, jnp.int32)
                    svec = f0c.at[jvec].get(mode="promise_in_bounds")
                    c = jvec + (chunk * LANES)
                    return tuple(
                        accs[g]
                        + plsc.load_gather(rows, [kiota + g * LANES, c])
                        * svec
                        for g in range(NGROUPS)
                    )

                accs = lax.fori_loop(0, LANES, c_body, accs, unroll=True)

            sims = [a * SCALE for a in accs]
            m = jnp.max(jnp.maximum(jnp.maximum(sims[0], sims[1]),
                                    jnp.maximum(sims[2], sims[3])))
            es = [jnp.exp(sv - m) for sv in sims]
            ssum = jnp.sum((es[0] + es[1]) + (es[2] + es[3]))
            invv = 1.0 / jnp.full((LANES,), ssum)
            mkey = BIG_KEY
            for g in range(NGROUPS):
                iv = idxb[pl.ds(r * K + g * LANES, LANES)]
                key = ((kiota + g * LANES) << IDX_BITS) | iv
                mkey = jnp.minimum(mkey,
                                   jnp.min(jnp.where(sims[g] >= m, key,
                                                     BIG_KEY)))
            nidx = mkey & ((1 << IDX_BITS) - 1)
            if store_conf:
                for g in range(NGROUPS):
                    confb[r, pl.ds(g * LANES, LANES)] = es[g] * invv
            rvec = jnp.full((LANES,), r, jnp.int32)
            plsc.store_scatter(ncb, [rvec], invv, mask=lane0)
            plsc.store_scatter(nib, [rvec], jnp.full((LANES,), nidx),
                               mask=lane0)

        pltpu.async_copy(fc_hbm.at[idxb.at[pl.ds(0, K)]], rows0, sem0)

        def pair_body(r2, carry):
            r = r2 * 2
            d1 = pltpu.async_copy(
                fc_hbm.at[idxb.at[pl.ds((r + 1) * K, K)]], rows1, sem1)
            pltpu.make_async_copy(
                fc_hbm.at[idxb.at[pl.ds(r * K, K)]], rows0, sem0).wait()
            compute_row(r, rows0)

            @pl.when(r2 < RPW // 2 - 1)
            def _():
                pltpu.async_copy(
                    fc_hbm.at[idxb.at[pl.ds((r + 2) * K, K)]], rows0, sem0)

            d1.wait()
            compute_row(r + 1, rows1)
            return carry

        lax.fori_loop(0, RPW // 2, pair_body, 0)

    run_direction(f0_hbm, f1_hbm, i01_hbm, True)
    pltpu.sync_copy(confb, conf_hbm.at[pl.ds(base, RPW)])
    pltpu.sync_copy(ncb, nc01_hbm.at[pl.ds(base, RPW)])
    pltpu.sync_copy(nib, ni01_hbm.at[pl.ds(base, RPW)])

    run_direction(f1_hbm, f0_hbm, i10_hbm, False)
    pltpu.sync_copy(ncb, nc10_hbm.at[pl.ds(base, RPW)])
    pltpu.sync_copy(nib, ni10_hbm.at[pl.ds(base, RPW)])


_match = pl.kernel(
    _match_body,
    out_type=[
        jax.ShapeDtypeStruct((L, K), jnp.float32),
        jax.ShapeDtypeStruct((L,), jnp.float32),
        jax.ShapeDtypeStruct((L,), jnp.int32),
        jax.ShapeDtypeStruct((L,), jnp.float32),
        jax.ShapeDtypeStruct((L,), jnp.int32),
    ],
    mesh=_mesh,
    scratch_types=[
        pltpu.VMEM((RPW * K,), jnp.int32),
        pltpu.VMEM((RPW, C), jnp.float32),
        pltpu.VMEM((K, C), jnp.float32),
        pltpu.VMEM((K, C), jnp.float32),
        pltpu.VMEM((RPW, K), jnp.float32),
        pltpu.VMEM((RPW,), jnp.float32),
        pltpu.VMEM((RPW,), jnp.int32),
        pltpu.SemaphoreType.DMA,
        pltpu.SemaphoreType.DMA,
    ],
    compiler_params=pltpu.CompilerParams(needs_layout_passes=False),
)


def _mutual_body(nc01_hbm, ni01_hbm, ni10_hbm, mconf_hbm,
                 ni10_full, ni01b, nc01b, mb):
    wid = lax.axis_index("s") * NC + lax.axis_index("c")
    base = wid * RPW
    kiota = lax.iota(jnp.int32, LANES)

    pltpu.sync_copy(ni10_hbm, ni10_full)
    pltpu.sync_copy(ni01_hbm.at[pl.ds(base, RPW)], ni01b)
    pltpu.sync_copy(nc01_hbm.at[pl.ds(base, RPW)], nc01b)

    def j_body(j, carry):
        off = j * LANES
        idxv = ni01b[pl.ds(off, LANES)]
        back = plsc.load_gather(ni10_full, [idxv])
        cv = nc01b[pl.ds(off, LANES)]
        lvec = base + off + kiota
        mask = (cv > INV_K) & (back == lvec)
        mb[pl.ds(off, LANES)] = jnp.where(mask, cv, 0.0)
        return carry

    lax.fori_loop(0, RPW // LANES, j_body, 0)
    pltpu.sync_copy(mb, mconf_hbm.at[pl.ds(base, RPW)])


_mutual = pl.kernel(
    _mutual_body,
    out_type=[jax.ShapeDtypeStruct((L,), jnp.float32)],
    mesh=_mesh,
    scratch_types=[
        pltpu.VMEM((L,), jnp.int32),
        pltpu.VMEM((RPW,), jnp.int32),
        pltpu.VMEM((RPW,), jnp.float32),
        pltpu.VMEM((RPW,), jnp.float32),
    ],
    compiler_params=pltpu.CompilerParams(needs_layout_passes=False),
)


def kernel(feat_c0, feat_c1, idx_c01, idx_c10):
    f0 = lax.optimization_barrier(
        (feat_c0[0] / (C ** 0.5)).astype(jnp.bfloat16)).astype(jnp.float32)
    f1 = lax.optimization_barrier(
        (feat_c1[0] / (C ** 0.5)).astype(jnp.bfloat16)).astype(jnp.float32)
    i01 = idx_c01[0].reshape(L * K)
    i10 = idx_c10[0].reshape(L * K)
    conf, nc01, ni01, nc10, ni10 = _match(f0, f1, i01, i10)
    (mconf,) = _mutual(nc01, ni01, ni10)
    return (conf[None], nc01[None], ni01[None], nc10[None], ni10[None],
            mconf[None])


# speedup vs baseline: 5.8767x; 1.1580x over previous
import functools

import jax
import jax.numpy as jnp
from jax import lax
from jax.experimental import pallas as pl
from jax.experimental.pallas import tpu as pltpu
from jax.experimental.pallas import tpu_sc as plsc

L = 9216
C = 128
K = 64
LANES = 16
NC, NS = 2, 16
NW = NC * NS
RPW = L // NW
SCALE = 10.0
INV_K = 1.0 / K
NGROUPS = K // LANES
CCHUNKS = C // LANES
IDX_BITS = 14
BIG_KEY = 1 << 22
NBUF = 2
RPD = 2
NPAIR = RPW // RPD

_mesh = plsc.VectorSubcoreMesh(
    core_axis_name="c", subcore_axis_name="s", num_cores=NC, num_subcores=NS
)


def _dir_body(store_conf, fq_hbm, fc_hbm, idx_hbm,
              conf_hbm, nc_hbm, ni_hbm,
              idxb, fqb, b0, b1, confb, ncb, nib, s0, s1):
    wid = lax.axis_index("s") * NC + lax.axis_index("c")
    base = wid * RPW
    kiota = lax.iota(jnp.int32, LANES)
    lane0 = kiota == 0
    bufs = [b0, b1]
    sems = [s0, s1]

    pltpu.sync_copy(idx_hbm.at[pl.ds(base * K, RPW * K)], idxb)
    pltpu.sync_copy(fq_hbm.at[pl.ds(base, RPW)], fqb)

    def fire(slot, pair):
        pltpu.async_copy(
            fc_hbm.at[idxb.at[pl.ds(pair * RPD * K, RPD * K)]],
            bufs[slot], sems[slot])

    def wait(slot, pair):
        pltpu.make_async_copy(
            fc_hbm.at[idxb.at[pl.ds(pair * RPD * K, RPD * K)]],
            bufs[slot], sems[slot]).wait()

    def compute_row(r, rows, koff):
        accs = tuple(jnp.zeros((LANES,), jnp.float32)
                     for _ in range(NGROUPS))
        for chunk in range(CCHUNKS):
            f0c = fqb[r, pl.ds(chunk * LANES, LANES)]

            def c_body(j, accs, f0c=f0c, chunk=chunk):
                jvec = jnp.full((LANES,), j, jnp.int32)
                svec = f0c.at[jvec].get(mode="promise_in_bounds")
                c = jvec + (chunk * LANES)
                return tuple(
                    accs[g]
                    + plsc.load_gather(
                        rows, [kiota + (koff + g * LANES), c]) * svec
                    for g in range(NGROUPS)
                )

            accs = lax.fori_loop(0, LANES, c_body, accs, unroll=4)

        sims = [a * SCALE for a in accs]
        m = jnp.max(jnp.maximum(jnp.maximum(sims[0], sims[1]),
                                jnp.maximum(sims[2], sims[3])))
        es = [jnp.exp(sv - m) for sv in sims]
        ssum = jnp.sum((es[0] + es[1]) + (es[2] + es[3]))
        invv = 1.0 / jnp.full((LANES,), ssum)
        mkey = BIG_KEY
        for g in range(NGROUPS):
            iv = idxb[pl.ds(r * K + g * LANES, LANES)]
            key = ((kiota + g * LANES) << IDX_BITS) | iv
            mkey = jnp.minimum(mkey,
                               jnp.min(jnp.where(sims[g] >= m, key,
                                                 BIG_KEY)))
        nidx = mkey & ((1 << IDX_BITS) - 1)
        if store_conf:
            for g in range(NGROUPS):
                confb[r, pl.ds(g * LANES, LANES)] = es[g] * invv
        rvec = jnp.full((LANES,), r, jnp.int32)
        plsc.store_scatter(ncb, [rvec], invv, mask=lane0)
        plsc.store_scatter(nib, [rvec], jnp.full((LANES,), nidx),
                           mask=lane0)

    for s in range(NBUF):
        fire(s, s)

    def ring_body(it, carry):
        for s in range(NBUF):
            pair = it * NBUF + s
            r = pair * RPD
            wait(s, pair)
            compute_row(r, bufs[s], 0)
            compute_row(r + 1, bufs[s], K)

            @pl.when(pair + NBUF < NPAIR)
            def _(pair=pair, s=s):
                fire(s, pair + NBUF)

        return carry

    lax.fori_loop(0, NPAIR // NBUF, ring_body, 0)

    if store_conf:
        pltpu.sync_copy(confb, conf_hbm.at[pl.ds(base, RPW)])
    pltpu.sync_copy(ncb, nc_hbm.at[pl.ds(base, RPW)])
    pltpu.sync_copy(nib, ni_hbm.at[pl.ds(base, RPW)])


def _make_dir_kernel(store_conf):
    out_type = [
        jax.ShapeDtypeStruct((L,), jnp.float32),
        jax.ShapeDtypeStruct((L,), jnp.int32),
    ]
    if store_conf:
        out_type = [jax.ShapeDtypeStruct((L, K), jnp.float32)] + out_type

    def body(fq_hbm, fc_hbm, idx_hbm, *rest):
        if store_conf:
            conf_hbm, nc_hbm, ni_hbm = rest[:3]
            scratch = rest[3:]
        else:
            nc_hbm, ni_hbm = rest[:2]
            conf_hbm = None
            scratch = rest[2:]
        _dir_body(store_conf, fq_hbm, fc_hbm, idx_hbm,
                  conf_hbm, nc_hbm, ni_hbm, *scratch)

    return pl.kernel(
        body,
        out_type=out_type,
        mesh=_mesh,
        scratch_types=[
            pltpu.VMEM((RPW * K,), jnp.int32),
            pltpu.VMEM((RPW, C), jnp.float32),
            pltpu.VMEM((RPD * K, C), jnp.float32),
            pltpu.VMEM((RPD * K, C), jnp.float32),
            pltpu.VMEM((RPW, K), jnp.float32),
            pltpu.VMEM((RPW,), jnp.float32),
            pltpu.VMEM((RPW,), jnp.int32),
            pltpu.SemaphoreType.DMA,
            pltpu.SemaphoreType.DMA,
        ],
        compiler_params=pltpu.CompilerParams(needs_layout_passes=False),
    )


_match01 = _make_dir_kernel(True)
_match10 = _make_dir_kernel(False)


def _mutual_body(nc01_hbm, ni01_hbm, ni10_hbm, mconf_hbm,
                 ni10_full, ni01b, nc01b, mb):
    wid = lax.axis_index("s") * NC + lax.axis_index("c")
    base = wid * RPW
    kiota = lax.iota(jnp.int32, LANES)

    pltpu.sync_copy(ni10_hbm, ni10_full)
    pltpu.sync_copy(ni01_hbm.at[pl.ds(base, RPW)], ni01b)
    pltpu.sync_copy(nc01_hbm.at[pl.ds(base, RPW)], nc01b)

    def j_body(j, carry):
        off = j * LANES
        idxv = ni01b[pl.ds(off, LANES)]
        back = plsc.load_gather(ni10_full, [idxv])
        cv = nc01b[pl.ds(off, LANES)]
        lvec = base + off + kiota
        mask = (cv > INV_K) & (back == lvec)
        mb[pl.ds(off, LANES)] = jnp.where(mask, cv, 0.0)
        return carry

    lax.fori_loop(0, RPW // LANES, j_body, 0)
    pltpu.sync_copy(mb, mconf_hbm.at[pl.ds(base, RPW)])


_mutual = pl.kernel(
    _mutual_body,
    out_type=[jax.ShapeDtypeStruct((L,), jnp.float32)],
    mesh=_mesh,
    scratch_types=[
        pltpu.VMEM((L,), jnp.int32),
        pltpu.VMEM((RPW,), jnp.int32),
        pltpu.VMEM((RPW,), jnp.float32),
        pltpu.VMEM((RPW,), jnp.float32),
    ],
    compiler_params=pltpu.CompilerParams(needs_layout_passes=False),
)


def kernel(feat_c0, feat_c1, idx_c01, idx_c10):
    f0 = lax.optimization_barrier(
        (feat_c0[0] / (C ** 0.5)).astype(jnp.bfloat16)).astype(jnp.float32)
    f1 = lax.optimization_barrier(
        (feat_c1[0] / (C ** 0.5)).astype(jnp.bfloat16)).astype(jnp.float32)
    i01 = idx_c01[0].reshape(L * K)
    i10 = idx_c10[0].reshape(L * K)
    conf, nc01, ni01 = _match01(f0, f1, i01)
    nc10, ni10 = _match10(f1, f0, i10)
    (mconf,) = _mutual(nc01, ni01, ni10)
    return (conf[None], nc01[None], ni01[None], nc10[None], ni10[None],
            mconf[None])


# speedup vs baseline: 9.1601x; 1.5587x over previous
import functools

import jax
import jax.numpy as jnp
from jax import lax
from jax.experimental import pallas as pl
from jax.experimental.pallas import tpu as pltpu
from jax.experimental.pallas import tpu_sc as plsc

L = 9216
C = 128
K = 64
LANES = 16
NC, NS = 2, 16
NW = NC * NS
RPW = L // NW
SCALE = 10.0
INV_K = 1.0 / K
NGROUPS = K // LANES
CCHUNKS = C // LANES
W = C // 2
IDX_BITS = 14
BIG_KEY = 1 << 22
NBUF = 2
RPD = 2
NPAIR = RPW // RPD

_mesh = plsc.VectorSubcoreMesh(
    core_axis_name="c", subcore_axis_name="s", num_cores=NC, num_subcores=NS
)


def _dir_body(store_conf, fq_hbm, fcw_hbm, idx_hbm,
              conf_hbm, nc_hbm, ni_hbm,
              idxb, fqb, b0, b1, confb, ncb, nib, s0, s1):
    wid = lax.axis_index("s") * NC + lax.axis_index("c")
    base = wid * RPW
    kiota = lax.iota(jnp.int32, LANES)
    lane0 = kiota == 0
    bufs = [b0, b1]
    sems = [s0, s1]

    pltpu.sync_copy(idx_hbm.at[pl.ds(base * K, RPW * K)], idxb)
    pltpu.sync_copy(fq_hbm.at[pl.ds(base, RPW)], fqb)

    def fire(slot, pair):
        pltpu.async_copy(
            fcw_hbm.at[idxb.at[pl.ds(pair * RPD * K, RPD * K)]],
            bufs[slot], sems[slot])

    def wait(slot, pair):
        pltpu.make_async_copy(
            fcw_hbm.at[idxb.at[pl.ds(pair * RPD * K, RPD * K)]],
            bufs[slot], sems[slot]).wait()

    def compute_row(r, rows, koff):
        accs = [jnp.zeros((LANES,), jnp.float32) for _ in range(NGROUPS)]
        kidx = [kiota + (koff + g * LANES) for g in range(NGROUPS)]
        for chunk in range(W // LANES):
            f0cA = fqb[r, pl.ds(chunk * 2 * LANES, LANES)]
            f0cB = fqb[r, pl.ds(chunk * 2 * LANES + LANES, LANES)]
            for j in range(LANES):
                w = chunk * LANES + j
                srcv = f0cA if j < LANES // 2 else f0cB
                sv0 = srcv.at[jnp.full((LANES,), (2 * j) % LANES,
                                       jnp.int32)].get(
                    mode="promise_in_bounds")
                sv1 = srcv.at[jnp.full((LANES,), (2 * j + 1) % LANES,
                                       jnp.int32)].get(
                    mode="promise_in_bounds")
                wv = jnp.full((LANES,), w, jnp.int32)
                for g in range(NGROUPS):
                    word = plsc.load_gather(rows, [kidx[g], wv])
                    lo = plsc.bitcast(word << 16, jnp.float32)
                    hi = plsc.bitcast(word & -65536, jnp.float32)
                    accs[g] = accs[g] + lo * sv0 + hi * sv1

        sims = [a * SCALE for a in accs]
        m = jnp.max(jnp.maximum(jnp.maximum(sims[0], sims[1]),
                                jnp.maximum(sims[2], sims[3])))
        es = [jnp.exp(sv - m) for sv in sims]
        ssum = jnp.sum((es[0] + es[1]) + (es[2] + es[3]))
        invv = 1.0 / jnp.full((LANES,), ssum)
        mkey = BIG_KEY
        for g in range(NGROUPS):
            iv = idxb[pl.ds(r * K + g * LANES, LANES)]
            key = ((kiota + g * LANES) << IDX_BITS) | iv
            mkey = jnp.minimum(mkey,
                               jnp.min(jnp.where(sims[g] >= m, key,
                                                 BIG_KEY)))
        nidx = mkey & ((1 << IDX_BITS) - 1)
        if store_conf:
            for g in range(NGROUPS):
                confb[r, pl.ds(g * LANES, LANES)] = es[g] * invv
        rvec = jnp.full((LANES,), r, jnp.int32)
        plsc.store_scatter(ncb, [rvec], invv, mask=lane0)
        plsc.store_scatter(nib, [rvec], jnp.full((LANES,), nidx),
                           mask=lane0)

    for s in range(NBUF):
        fire(s, s)

    def ring_body(it, carry):
        for s in range(NBUF):
            pair = it * NBUF + s
            r = pair * RPD
            wait(s, pair)
            compute_row(r, bufs[s], 0)
            compute_row(r + 1, bufs[s], K)

            @pl.when(pair + NBUF < NPAIR)
            def _(pair=pair, s=s):
                fire(s, pair + NBUF)

        return carry

    lax.fori_loop(0, NPAIR // NBUF, ring_body, 0)

    if store_conf:
        pltpu.sync_copy(confb, conf_hbm.at[pl.ds(base, RPW)])
    pltpu.sync_copy(ncb, nc_hbm.at[pl.ds(base, RPW)])
    pltpu.sync_copy(nib, ni_hbm.at[pl.ds(base, RPW)])


def _make_dir_kernel(store_conf):
    out_type = [
        jax.ShapeDtypeStruct((L,), jnp.float32),
        jax.ShapeDtypeStruct((L,), jnp.int32),
    ]
    if store_conf:
        out_type = [jax.ShapeDtypeStruct((L, K), jnp.float32)] + out_type

    def body(fq_hbm, fc_hbm, idx_hbm, *rest):
        if store_conf:
            conf_hbm, nc_hbm, ni_hbm = rest[:3]
            scratch = rest[3:]
        else:
            nc_hbm, ni_hbm = rest[:2]
            conf_hbm = None
            scratch = rest[2:]
        _dir_body(store_conf, fq_hbm, fc_hbm, idx_hbm,
                  conf_hbm, nc_hbm, ni_hbm, *scratch)

    return pl.kernel(
        body,
        out_type=out_type,
        mesh=_mesh,
        scratch_types=[
            pltpu.VMEM((RPW * K,), jnp.int32),
            pltpu.VMEM((RPW, C), jnp.float32),
            pltpu.VMEM((RPD * K, W), jnp.int32),
            pltpu.VMEM((RPD * K, W), jnp.int32),
            pltpu.VMEM((RPW, K), jnp.float32),
            pltpu.VMEM((RPW,), jnp.float32),
            pltpu.VMEM((RPW,), jnp.int32),
            pltpu.SemaphoreType.DMA,
            pltpu.SemaphoreType.DMA,
        ],
        compiler_params=pltpu.CompilerParams(
            needs_layout_passes=False, use_tc_tiling_on_sc=False),
    )


_match01 = _make_dir_kernel(True)
_match10 = _make_dir_kernel(False)


def _mutual_body(nc01_hbm, ni01_hbm, ni10_hbm, mconf_hbm,
                 ni10_full, ni01b, nc01b, mb):
    wid = lax.axis_index("s") * NC + lax.axis_index("c")
    base = wid * RPW
    kiota = lax.iota(jnp.int32, LANES)

    pltpu.sync_copy(ni10_hbm, ni10_full)
    pltpu.sync_copy(ni01_hbm.at[pl.ds(base, RPW)], ni01b)
    pltpu.sync_copy(nc01_hbm.at[pl.ds(base, RPW)], nc01b)

    def j_body(j, carry):
        off = j * LANES
        idxv = ni01b[pl.ds(off, LANES)]
        back = plsc.load_gather(ni10_full, [idxv])
        cv = nc01b[pl.ds(off, LANES)]
        lvec = base + off + kiota
        mask = (cv > INV_K) & (back == lvec)
        mb[pl.ds(off, LANES)] = jnp.where(mask, cv, 0.0)
        return carry

    lax.fori_loop(0, RPW // LANES, j_body, 0)
    pltpu.sync_copy(mb, mconf_hbm.at[pl.ds(base, RPW)])


_mutual = pl.kernel(
    _mutual_body,
    out_type=[jax.ShapeDtypeStruct((L,), jnp.float32)],
    mesh=_mesh,
    scratch_types=[
        pltpu.VMEM((L,), jnp.int32),
        pltpu.VMEM((RPW,), jnp.int32),
        pltpu.VMEM((RPW,), jnp.float32),
        pltpu.VMEM((RPW,), jnp.float32),
    ],
    compiler_params=pltpu.CompilerParams(needs_layout_passes=False),
)


def kernel(feat_c0, feat_c1, idx_c01, idx_c10):
    f0b = lax.optimization_barrier(
        (feat_c0[0] / (C ** 0.5)).astype(jnp.bfloat16))
    f1b = lax.optimization_barrier(
        (feat_c1[0] / (C ** 0.5)).astype(jnp.bfloat16))
    f0 = f0b.astype(jnp.float32)
    f1 = f1b.astype(jnp.float32)
    f0w = lax.bitcast_convert_type(f0b.reshape(L, W, 2), jnp.int32)
    f1w = lax.bitcast_convert_type(f1b.reshape(L, W, 2), jnp.int32)
    i01 = idx_c01[0].reshape(L * K)
    i10 = idx_c10[0].reshape(L * K)
    conf, nc01, ni01 = _match01(f0, f1w, i01)
    nc10, ni10 = _match10(f1, f0w, i10)
    (mconf,) = _mutual(nc01, ni01, ni10)
    return (conf[None], nc01[None], ni01[None], nc10[None], ni10[None],
            mconf[None])
